# baseline (speedup 1.0000x reference)
import functools

import jax
import jax.numpy as jnp
from jax import lax
from jax.experimental import pallas as pl
from jax.experimental.pallas import tpu as pltpu
from jax.experimental.pallas import tpu_sc as plsc

N = 10000
E = 160000
DV = 256
DE = 16
DU = 32
H = 256
VOUT = 256
EOUT = 128
UOUT = 32

CHUNK = 128
NW = 32
N_PAD = 10240
ROWS_PER_TILE = N_PAD // 16

NCHUNKS = E // CHUNK
SITER = (NCHUNKS + NW - 1) // NW
NCH_PAD = SITER * NW

E_S = E // 2
NCH_S = E_S // CHUNK
GITER = (NCH_S + NW - 1) // NW

_mesh = plsc.VectorSubcoreMesh(core_axis_name="c", subcore_axis_name="s")


def _ln(h, g, b):
    mu = jnp.mean(h, axis=-1, keepdims=True)
    d = h - mu
    var = jnp.mean(d * d, axis=-1, keepdims=True)
    return d * jax.lax.rsqrt(var + 1e-5) * g + b


def _pack_bf16(block):
    b = jax.lax.bitcast_convert_type(block, jnp.uint32)
    r = lambda v: (v + 0x7FFF + ((v >> 16) & 1)) >> 16
    return r(b[:, :H // 2]) | (r(b[:, H // 2:]) << 16)


def _unpack_bf16(packed):
    left = jax.lax.bitcast_convert_type(packed << 16, jnp.float32)
    right = jax.lax.bitcast_convert_type(packed & jnp.uint32(0xFFFF0000),
                                         jnp.float32)
    return left, right


def _pre_body(x_ref, w_ref, u_ref, wu2_ref, b2_ref, xr_ref, xc_ref, xn_ref,
              c1_ref, c2_ref):
    prod = jnp.dot(x_ref[...].astype(jnp.bfloat16), w_ref[...],
                   preferred_element_type=jnp.float32)
    xr_ref[...] = _pack_bf16(prod[:, :H])
    xc_ref[...] = _pack_bf16(prod[:, H:2 * H])
    xn_ref[...] = prod[:, 2 * H:]

    @pl.when(pl.program_id(0) == 0)
    def _():
        cu = jnp.dot(u_ref[...], wu2_ref[...],
                     preferred_element_type=jnp.float32) + b2_ref[...]
        c1_ref[...] = cu[:, :H]
        c2_ref[...] = cu[:, H:]


def _precompute(x, wrcx, u, wu2, b2):
    bn = 1000
    return pl.pallas_call(
        _pre_body,
        grid=(N // bn,),
        in_specs=[
            pl.BlockSpec((bn, DV), lambda i: (i, 0)),
            pl.BlockSpec((DV, 3 * H), lambda i: (0, 0)),
            pl.BlockSpec((1, DU), lambda i: (0, 0)),
            pl.BlockSpec((DU, 2 * H), lambda i: (0, 0)),
            pl.BlockSpec((1, 2 * H), lambda i: (0, 0)),
        ],
        out_specs=[
            pl.BlockSpec((bn, H // 2), lambda i: (i, 0)),
            pl.BlockSpec((bn, H // 2), lambda i: (i, 0)),
            pl.BlockSpec((bn, H), lambda i: (i, 0)),
            pl.BlockSpec((1, H), lambda i: (0, 0)),
            pl.BlockSpec((1, H), lambda i: (0, 0)),
        ],
        out_shape=[jax.ShapeDtypeStruct((N, H // 2), jnp.uint32),
                   jax.ShapeDtypeStruct((N, H // 2), jnp.uint32),
                   jax.ShapeDtypeStruct((N, H), jnp.float32),
                   jax.ShapeDtypeStruct((1, H), jnp.float32),
                   jax.ShapeDtypeStruct((1, H), jnp.float32)],
    )(x, wrcx, u, wu2, b2)


def _gather_body(xr_hbm, xc_hbm, row_hbm, col_hbm, gr_hbm, gc_hbm,
                 idx_r, idx_c, buf_r, buf_c, sem_r, sem_c):
    wid = lax.axis_index("s") * 2 + lax.axis_index("c")

    def step(i, _):
        c = jnp.minimum(wid + i * NW, NCH_S - 1)
        pltpu.sync_copy(row_hbm.at[c], idx_r)
        pltpu.sync_copy(col_hbm.at[c], idx_c)
        cp_r = pltpu.async_copy(xr_hbm.at[idx_r], buf_r, sem_r)
        cp_c = pltpu.async_copy(xc_hbm.at[idx_c], buf_c, sem_c)
        cp_r.wait()
        cp_c.wait()
        pltpu.sync_copy(buf_r, gr_hbm.at[pl.ds(c * CHUNK, CHUNK)])
        pltpu.sync_copy(buf_c, gc_hbm.at[pl.ds(c * CHUNK, CHUNK)])
        return 0

    lax.fori_loop(0, GITER, step, 0)


@functools.partial(
    pl.kernel,
    out_type=[jax.ShapeDtypeStruct((E_S, H // 2), jnp.uint32),
              jax.ShapeDtypeStruct((E_S, H // 2), jnp.uint32)],
    mesh=_mesh,
    scratch_types=[
        pltpu.VMEM((CHUNK,), jnp.int32),
        pltpu.VMEM((CHUNK,), jnp.int32),
        pltpu.VMEM((CHUNK, H // 2), jnp.uint32),
        pltpu.VMEM((CHUNK, H // 2), jnp.uint32),
        pltpu.SemaphoreType.DMA,
        pltpu.SemaphoreType.DMA,
    ],
)
def _gather(*args):
    _gather_body(*args)


def _edge_body(gr_ref, gc_ref, ea_ref, wa_ref,
               ew2_ref, eb2_ref, eg_ref, ebt_ref, ne_ref, esum_ref):
    i = pl.program_id(0)
    rl, rr = _unpack_bf16(gr_ref[...])
    cl, cr = _unpack_bf16(gc_ref[...])
    h = jnp.concatenate([rl + cl, rr + cr], axis=1)
    h = h + jnp.dot(ea_ref[...], wa_ref[...], preferred_element_type=jnp.float32)
    h = jnp.maximum(h, 0.0)
    h = jnp.dot(h.astype(jnp.bfloat16), ew2_ref[...],
                preferred_element_type=jnp.float32) + eb2_ref[...]
    h = jnp.maximum(h, 0.0)
    ne = _ln(h, eg_ref[...], ebt_ref[...])
    ne_ref[...] = ne

    @pl.when(i == 0)
    def _():
        esum_ref[...] = jnp.zeros_like(esum_ref)
    esum_ref[...] += jnp.sum(ne, axis=0, keepdims=True)


def _edge_mlp(gr, gc, ea_aug, wa_aug, ew2, eb2, eg, ebt):
    be = 4000
    return pl.pallas_call(
        _edge_body,
        grid=(E_S // be,),
        in_specs=[
            pl.BlockSpec((be, H // 2), lambda i: (i, 0)),
            pl.BlockSpec((be, H // 2), lambda i: (i, 0)),
            pl.BlockSpec((be, DE + 1), lambda i: (i, 0)),
            pl.BlockSpec((DE + 1, H), lambda i: (0, 0)),
            pl.BlockSpec((H, EOUT), lambda i: (0, 0)),
            pl.BlockSpec((1, EOUT), lambda i: (0, 0)),
            pl.BlockSpec((1, EOUT), lambda i: (0, 0)),
            pl.BlockSpec((1, EOUT), lambda i: (0, 0)),
        ],
        out_specs=[
            pl.BlockSpec((be, EOUT), lambda i: (i, 0)),
            pl.BlockSpec((1, EOUT), lambda i: (0, 0)),
        ],
        out_shape=[jax.ShapeDtypeStruct((E_S, EOUT), jnp.float32),
                   jax.ShapeDtypeStruct((1, EOUT), jnp.float32)],
    )(gr, gc, ea_aug, wa_aug, ew2, eb2, eg, ebt)


def _scatter_body(nea_hbm, neb_hbm, row_hbm, agg_hbm, acc, idx_a, idx_b,
                  buf_a, buf_b, sem_a, sem_b):
    cid = lax.axis_index("c")
    sid = lax.axis_index("s")
    wid = sid * 2 + cid
    npass = ROWS_PER_TILE // CHUNK

    zeros16 = jnp.zeros((16,), jnp.float32)

    def zstep(r, _):
        for j in range(EOUT // 16):
            buf_a[r, pl.ds(j * 16, 16)] = zeros16
        return 0

    lax.fori_loop(0, CHUNK, zstep, 0)
    for p in range(npass):
        pltpu.sync_copy(
            buf_a, acc.at[pl.ds(sid * ROWS_PER_TILE + p * CHUNK, CHUNK)])
    plsc.subcore_barrier()

    def load(c, idx, buf, sem):
        pltpu.sync_copy(row_hbm.at[c], idx)

        @pl.when(c < NCH_S)
        def _():
            pltpu.async_copy(nea_hbm.at[pl.ds(c * CHUNK, CHUNK)], buf, sem)

        @pl.when(c >= NCH_S)
        def _():
            c_data = jnp.minimum(c - NCH_S, NCH_S - 1)
            pltpu.async_copy(
                neb_hbm.at[pl.ds(c_data * CHUNK, CHUNK)], buf, sem)

    load(wid, idx_a, buf_a, sem_a)
    load(wid + NW, idx_b, buf_b, sem_b)

    def step(p, _):
        ca = wid + 2 * p * NW
        pltpu.make_async_copy(nea_hbm.at[pl.ds(0, CHUNK)], buf_a, sem_a).wait()
        pltpu.sync_copy(buf_a, acc.at[idx_a], add=True)

        @pl.when(2 * p + 2 < SITER)
        def _():
            load(ca + 2 * NW, idx_a, buf_a, sem_a)
        pltpu.make_async_copy(nea_hbm.at[pl.ds(0, CHUNK)], buf_b, sem_b).wait()
        pltpu.sync_copy(buf_b, acc.at[idx_b], add=True)

        @pl.when(2 * p + 3 < SITER)
        def _():
            load(ca + 3 * NW, idx_b, buf_b, sem_b)
        return 0

    lax.fori_loop(0, SITER // 2, step, 0)
    plsc.subcore_barrier()
    for p in range(npass):
        base = sid * ROWS_PER_TILE + p * CHUNK
        pltpu.sync_copy(acc.at[pl.ds(base, CHUNK)], buf_a)
        pltpu.sync_copy(buf_a, agg_hbm.at[cid, pl.ds(base, CHUNK)])


@functools.partial(
    pl.kernel,
    out_type=jax.ShapeDtypeStruct((2, N_PAD, EOUT), jnp.float32),
    mesh=_mesh,
    scratch_types=[
        pltpu.VMEM_SHARED((N_PAD, EOUT), jnp.float32),
        pltpu.VMEM((CHUNK,), jnp.int32),
        pltpu.VMEM((CHUNK,), jnp.int32),
        pltpu.VMEM((CHUNK, EOUT), jnp.float32),
        pltpu.VMEM((CHUNK, EOUT), jnp.float32),
        pltpu.SemaphoreType.DMA,
        pltpu.SemaphoreType.DMA,
    ],
)
def _scatter(*args):
    _scatter_body(*args)


def _node_body(xn_ref, a0_ref, a1_ref, wna_ref, c2_ref,
               nw2_ref, nb2_ref, ng_ref, nbt_ref, nx_ref, xsum_ref):
    i = pl.program_id(0)
    agg = (a0_ref[...] + a1_ref[...]).astype(jnp.bfloat16)
    h = xn_ref[...] + jnp.dot(agg, wna_ref[...],
                              preferred_element_type=jnp.float32) + c2_ref[...]
    h = jnp.maximum(h, 0.0)
    h = jnp.dot(h.astype(jnp.bfloat16), nw2_ref[...],
                preferred_element_type=jnp.float32) + nb2_ref[...]
    h = jnp.maximum(h, 0.0)
    nx = _ln(h, ng_ref[...], nbt_ref[...])
    nx_ref[...] = nx

    @pl.when(i == 0)
    def _():
        xsum_ref[...] = jnp.zeros_like(xsum_ref)
    xsum_ref[...] += jnp.sum(nx, axis=0, keepdims=True)


def _node_mlp(xn, a0, a1, wna, c2, nw2, nb2, ng, nbt):
    bn = 1000
    return pl.pallas_call(
        _node_body,
        grid=(N // bn,),
        in_specs=[
            pl.BlockSpec((bn, H), lambda i: (i, 0)),
            pl.BlockSpec((bn, EOUT), lambda i: (i, 0)),
            pl.BlockSpec((bn, EOUT), lambda i: (i, 0)),
            pl.BlockSpec((EOUT, H), lambda i: (0, 0)),
            pl.BlockSpec((1, H), lambda i: (0, 0)),
            pl.BlockSpec((H, VOUT), lambda i: (0, 0)),
            pl.BlockSpec((1, VOUT), lambda i: (0, 0)),
            pl.BlockSpec((1, VOUT), lambda i: (0, 0)),
            pl.BlockSpec((1, VOUT), lambda i: (0, 0)),
        ],
        out_specs=[
            pl.BlockSpec((bn, VOUT), lambda i: (i, 0)),
            pl.BlockSpec((1, VOUT), lambda i: (0, 0)),
        ],
        out_shape=[jax.ShapeDtypeStruct((N, VOUT), jnp.float32),
                   jax.ShapeDtypeStruct((1, VOUT), jnp.float32)],
    )(xn, a0, a1, wna, c2, nw2, nb2, ng, nbt)


def _global_body(u_ref, xsum_ref, esum_ref, gu_ref, gx_ref, ge_ref, gb1_ref,
                 gw2_ref, gb2_ref, gg_ref, gbt_ref, nu_ref):
    h = jnp.dot(u_ref[...], gu_ref[...], preferred_element_type=jnp.float32)
    h = h + jnp.dot(xsum_ref[...] * (1.0 / N), gx_ref[...],
                    preferred_element_type=jnp.float32)
    h = h + jnp.dot(esum_ref[...] * (1.0 / E), ge_ref[...],
                    preferred_element_type=jnp.float32)
    h = jnp.maximum(h + gb1_ref[...], 0.0)
    h = jnp.dot(h, gw2_ref[...], preferred_element_type=jnp.float32) + gb2_ref[...]
    h = jnp.maximum(h, 0.0)
    nu_ref[...] = _ln(h, gg_ref[...], gbt_ref[...])


def _global_mlp(u, xsum, esum, gu, gx, ge, gb1, gw2, gb2, gg, gbt):
    return pl.pallas_call(
        _global_body,
        out_shape=jax.ShapeDtypeStruct((1, UOUT), jnp.float32),
    )(u, xsum, esum, gu, gx, ge, gb1, gw2, gb2, gg, gbt)


def kernel(x, edge_index, edge_attr, u, v_indices, e_indices,
           ew1, eb1, ew2, eb2, eg, ebt,
           nw1, nb1, nw2, nb2, ng, nbt,
           gw1, gb1, gw2, gb2, gg, gbt):
    row = edge_index[0]
    col = edge_index[1]
    rows2d = [row[s * E_S:(s + 1) * E_S].reshape(NCH_S, CHUNK)
              for s in range(2)]
    cols2d = [col[s * E_S:(s + 1) * E_S].reshape(NCH_S, CHUNK)
              for s in range(2)]
    pad = jnp.full(((NCH_PAD - NCHUNKS) * CHUNK,), N, jnp.int32)
    rowpad2d = jnp.concatenate([row, pad]).reshape(NCH_PAD, CHUNK)

    bf = jnp.bfloat16
    wrcx = jnp.concatenate([ew1[:DV], ew1[DV:2 * DV], nw1[:DV]], axis=1).astype(bf)
    wu2 = jnp.concatenate([ew1[2 * DV + DE:], nw1[DV + EOUT:]], axis=1)
    b2 = jnp.concatenate([eb1, nb1]).reshape(1, -1)
    wna = nw1[DV:DV + EOUT].astype(bf)
    gu_w = gw1[:DU]
    gx_w = gw1[DU:DU + VOUT]
    ge_w = gw1[DU + VOUT:]
    ea_aug = jnp.concatenate(
        [edge_attr.astype(bf), jnp.ones((E, 1), bf)], axis=1)

    r2 = lambda v: v.reshape(1, -1)

    xr, xc, xn, c1, c2 = _precompute(x, wrcx, u, wu2, b2)
    wa_aug = jnp.concatenate(
        [ew1[2 * DV:2 * DV + DE], c1], axis=0).astype(bf)

    ew2_b = ew2.astype(bf)
    ne_s, esum_s = [], []
    for s in range(2):
        gr, gc = _gather(xr, xc, rows2d[s], cols2d[s])
        ne, es = _edge_mlp(gr, gc, ea_aug[s * E_S:(s + 1) * E_S], wa_aug,
                           ew2_b, r2(eb2), r2(eg), r2(ebt))
        ne_s.append(ne)
        esum_s.append(es)

    new_e = jnp.concatenate(ne_s, axis=0)
    esum = esum_s[0] + esum_s[1]
    aggp = _scatter(ne_s[0], ne_s[1], rowpad2d)
    new_x, xsum = _node_mlp(xn, aggp[0], aggp[1], wna, c2,
                            nw2.astype(bf), r2(nb2), r2(ng), r2(nbt))
    new_u = _global_mlp(u, xsum, esum, gu_w, gx_w, ge_w, r2(gb1),
                        gw2, r2(gb2), r2(gg), r2(gbt))
    return (new_x, new_e, new_u)


# speedup vs baseline: 1.0196x; 1.0196x over previous
import functools

import jax
import jax.numpy as jnp
from jax import lax
from jax.experimental import pallas as pl
from jax.experimental.pallas import tpu as pltpu
from jax.experimental.pallas import tpu_sc as plsc

N = 10000
E = 160000
DV = 256
DE = 16
DU = 32
H = 256
VOUT = 256
EOUT = 128
UOUT = 32

CHUNK = 128
NW = 32
N_PAD = 10240
ROWS_PER_TILE = N_PAD // 16

NCHUNKS = E // CHUNK
GITER = (NCHUNKS + 15) // 16
SITER = (NCHUNKS + NW - 1) // NW
NCH_PAD = SITER * NW
TLOAD = N // 1000

_mesh = plsc.VectorSubcoreMesh(core_axis_name="c", subcore_axis_name="s")


def _ln(h, g, b):
    mu = jnp.mean(h, axis=-1, keepdims=True)
    d = h - mu
    var = jnp.mean(d * d, axis=-1, keepdims=True)
    return d * jax.lax.rsqrt(var + 1e-5) * g + b


def _pack_bf16(block):
    b = jax.lax.bitcast_convert_type(block, jnp.uint32)
    r = lambda v: (v + 0x7FFF + ((v >> 16) & 1)) >> 16
    return r(b[:, :H // 2]) | (r(b[:, H // 2:]) << 16)


def _unpack_bf16(packed):
    left = jax.lax.bitcast_convert_type(packed << 16, jnp.float32)
    right = jax.lax.bitcast_convert_type(packed & jnp.uint32(0xFFFF0000),
                                         jnp.float32)
    return left, right


def _pre_body(x_ref, w_ref, u_ref, wu2_ref, b2_ref, xr_ref, xc_ref, xn_ref,
              c1_ref, c2_ref):
    prod = jnp.dot(x_ref[...].astype(jnp.bfloat16), w_ref[...],
                   preferred_element_type=jnp.float32)
    xr_ref[...] = _pack_bf16(prod[:, :H])
    xc_ref[...] = _pack_bf16(prod[:, H:2 * H])
    xn_ref[...] = prod[:, 2 * H:]

    @pl.when(pl.program_id(0) == 0)
    def _():
        cu = jnp.dot(u_ref[...], wu2_ref[...],
                     preferred_element_type=jnp.float32) + b2_ref[...]
        c1_ref[...] = cu[:, :H]
        c2_ref[...] = cu[:, H:]


def _precompute(x, wrcx, u, wu2, b2):
    bn = 1000
    return pl.pallas_call(
        _pre_body,
        grid=(N // bn,),
        in_specs=[
            pl.BlockSpec((bn, DV), lambda i: (i, 0)),
            pl.BlockSpec((DV, 3 * H), lambda i: (0, 0)),
            pl.BlockSpec((1, DU), lambda i: (0, 0)),
            pl.BlockSpec((DU, 2 * H), lambda i: (0, 0)),
            pl.BlockSpec((1, 2 * H), lambda i: (0, 0)),
        ],
        out_specs=[
            pl.BlockSpec((bn, H // 2), lambda i: (i, 0)),
            pl.BlockSpec((bn, H // 2), lambda i: (i, 0)),
            pl.BlockSpec((bn, H), lambda i: (i, 0)),
            pl.BlockSpec((1, H), lambda i: (0, 0)),
            pl.BlockSpec((1, H), lambda i: (0, 0)),
        ],
        out_shape=[jax.ShapeDtypeStruct((N, H // 2), jnp.uint32),
                   jax.ShapeDtypeStruct((N, H // 2), jnp.uint32),
                   jax.ShapeDtypeStruct((N, H), jnp.float32),
                   jax.ShapeDtypeStruct((1, H), jnp.float32),
                   jax.ShapeDtypeStruct((1, H), jnp.float32)],
    )(x, wrcx, u, wu2, b2)


def _gather_body(xr_hbm, xc_hbm, row_hbm, col_hbm, gr_hbm, gc_hbm,
                 idx_r, idx_c, buf_r, buf_c, sem_r, sem_c):
    wid = lax.axis_index("s") * 2 + lax.axis_index("c")

    def step(i, _):
        c = jnp.minimum(wid + i * NW, NCHUNKS - 1)
        pltpu.sync_copy(row_hbm.at[c], idx_r)
        pltpu.sync_copy(col_hbm.at[c], idx_c)
        cp_r = pltpu.async_copy(xr_hbm.at[idx_r], buf_r, sem_r)
        cp_c = pltpu.async_copy(xc_hbm.at[idx_c], buf_c, sem_c)
        cp_r.wait()
        cp_c.wait()
        pltpu.sync_copy(buf_r, gr_hbm.at[pl.ds(c * CHUNK, CHUNK)])
        pltpu.sync_copy(buf_c, gc_hbm.at[pl.ds(c * CHUNK, CHUNK)])
        return 0

    lax.fori_loop(0, SITER, step, 0)


@functools.partial(
    pl.kernel,
    out_type=[jax.ShapeDtypeStruct((E, H // 2), jnp.uint32),
              jax.ShapeDtypeStruct((E, H // 2), jnp.uint32)],
    mesh=_mesh,
    scratch_types=[
        pltpu.VMEM((CHUNK,), jnp.int32),
        pltpu.VMEM((CHUNK,), jnp.int32),
        pltpu.VMEM((CHUNK, H // 2), jnp.uint32),
        pltpu.VMEM((CHUNK, H // 2), jnp.uint32),
        pltpu.SemaphoreType.DMA,
        pltpu.SemaphoreType.DMA,
    ],
)
def _gather(*args):
    _gather_body(*args)


def _edge_body(gr_ref, gc_ref, ea_ref, wa_ref, c1_ref,
               ew2_ref, eb2_ref, eg_ref, ebt_ref, ne_ref, esum_ref):
    i = pl.program_id(0)
    rl, rr = _unpack_bf16(gr_ref[...])
    cl, cr = _unpack_bf16(gc_ref[...])
    h = jnp.concatenate([rl + cl, rr + cr], axis=1) + c1_ref[...]
    h = h + jnp.dot(ea_ref[...].astype(jnp.bfloat16), wa_ref[...],
                    preferred_element_type=jnp.float32)
    h = jnp.maximum(h, 0.0)
    h = jnp.dot(h.astype(jnp.bfloat16), ew2_ref[...],
                preferred_element_type=jnp.float32) + eb2_ref[...]
    h = jnp.maximum(h, 0.0)
    ne = _ln(h, eg_ref[...], ebt_ref[...])
    ne_ref[...] = ne

    @pl.when(i == 0)
    def _():
        esum_ref[...] = jnp.zeros_like(esum_ref)
    esum_ref[...] += jnp.sum(ne, axis=0, keepdims=True)


def _edge_mlp(gr, gc, ea, wa, c1, ew2, eb2, eg, ebt):
    be = 4000
    return pl.pallas_call(
        _edge_body,
        grid=(E // be,),
        in_specs=[
            pl.BlockSpec((be, H // 2), lambda i: (i, 0)),
            pl.BlockSpec((be, H // 2), lambda i: (i, 0)),
            pl.BlockSpec((be, DE), lambda i: (i, 0)),
            pl.BlockSpec((DE, H), lambda i: (0, 0)),
            pl.BlockSpec((1, H), lambda i: (0, 0)),
            pl.BlockSpec((H, EOUT), lambda i: (0, 0)),
            pl.BlockSpec((1, EOUT), lambda i: (0, 0)),
            pl.BlockSpec((1, EOUT), lambda i: (0, 0)),
            pl.BlockSpec((1, EOUT), lambda i: (0, 0)),
        ],
        out_specs=[
            pl.BlockSpec((be, EOUT), lambda i: (i, 0)),
            pl.BlockSpec((1, EOUT), lambda i: (0, 0)),
        ],
        out_shape=[jax.ShapeDtypeStruct((E, EOUT), jnp.float32),
                   jax.ShapeDtypeStruct((1, EOUT), jnp.float32)],
    )(gr, gc, ea, wa, c1, ew2, eb2, eg, ebt)


def _scatter_body(ne_hbm, row_hbm, agg_hbm, acc, idx_a, idx_b,
                  buf_a, buf_b, sem_a, sem_b):
    cid = lax.axis_index("c")
    sid = lax.axis_index("s")
    wid = sid * 2 + cid
    npass = ROWS_PER_TILE // CHUNK

    zeros16 = jnp.zeros((16,), jnp.float32)

    def zstep(r, _):
        for j in range(EOUT // 16):
            buf_a[r, pl.ds(j * 16, 16)] = zeros16
        return 0

    lax.fori_loop(0, CHUNK, zstep, 0)
    for p in range(npass):
        pltpu.sync_copy(
            buf_a, acc.at[pl.ds(sid * ROWS_PER_TILE + p * CHUNK, CHUNK)])
    plsc.subcore_barrier()

    def load(c, idx, buf, sem):
        pltpu.sync_copy(row_hbm.at[c], idx)
        c_data = jnp.minimum(c, NCHUNKS - 1)
        pltpu.async_copy(ne_hbm.at[pl.ds(c_data * CHUNK, CHUNK)], buf, sem)

    load(wid, idx_a, buf_a, sem_a)
    load(wid + NW, idx_b, buf_b, sem_b)

    def step(p, _):
        ca = wid + 2 * p * NW
        pltpu.make_async_copy(ne_hbm.at[pl.ds(0, CHUNK)], buf_a, sem_a).wait()
        pltpu.sync_copy(buf_a, acc.at[idx_a], add=True)

        @pl.when(2 * p + 2 < SITER)
        def _():
            load(ca + 2 * NW, idx_a, buf_a, sem_a)
        pltpu.make_async_copy(ne_hbm.at[pl.ds(0, CHUNK)], buf_b, sem_b).wait()
        pltpu.sync_copy(buf_b, acc.at[idx_b], add=True)

        @pl.when(2 * p + 3 < SITER)
        def _():
            load(ca + 3 * NW, idx_b, buf_b, sem_b)
        return 0

    lax.fori_loop(0, SITER // 2, step, 0)
    plsc.subcore_barrier()
    for p in range(npass):
        base = sid * ROWS_PER_TILE + p * CHUNK
        pltpu.sync_copy(acc.at[pl.ds(base, CHUNK)], buf_a)
        pltpu.sync_copy(buf_a, agg_hbm.at[cid, pl.ds(base, CHUNK)])


@functools.partial(
    pl.kernel,
    out_type=jax.ShapeDtypeStruct((2, N_PAD, EOUT), jnp.float32),
    mesh=_mesh,
    scratch_types=[
        pltpu.VMEM_SHARED((N_PAD, EOUT), jnp.float32),
        pltpu.VMEM((CHUNK,), jnp.int32),
        pltpu.VMEM((CHUNK,), jnp.int32),
        pltpu.VMEM((CHUNK, EOUT), jnp.float32),
        pltpu.VMEM((CHUNK, EOUT), jnp.float32),
        pltpu.SemaphoreType.DMA,
        pltpu.SemaphoreType.DMA,
    ],
)
def _scatter(*args):
    _scatter_body(*args)


def _node_body(xn_ref, a0_ref, a1_ref, wna_ref, c2_ref,
               nw2_ref, nb2_ref, ng_ref, nbt_ref, nx_ref, xsum_ref):
    i = pl.program_id(0)
    agg = (a0_ref[...] + a1_ref[...]).astype(jnp.bfloat16)
    h = xn_ref[...] + jnp.dot(agg, wna_ref[...],
                              preferred_element_type=jnp.float32) + c2_ref[...]
    h = jnp.maximum(h, 0.0)
    h = jnp.dot(h.astype(jnp.bfloat16), nw2_ref[...],
                preferred_element_type=jnp.float32) + nb2_ref[...]
    h = jnp.maximum(h, 0.0)
    nx = _ln(h, ng_ref[...], nbt_ref[...])
    nx_ref[...] = nx

    @pl.when(i == 0)
    def _():
        xsum_ref[...] = jnp.zeros_like(xsum_ref)
    xsum_ref[...] += jnp.sum(nx, axis=0, keepdims=True)


def _node_mlp(xn, a0, a1, wna, c2, nw2, nb2, ng, nbt):
    bn = 1000
    return pl.pallas_call(
        _node_body,
        grid=(N // bn,),
        in_specs=[
            pl.BlockSpec((bn, H), lambda i: (i, 0)),
            pl.BlockSpec((bn, EOUT), lambda i: (i, 0)),
            pl.BlockSpec((bn, EOUT), lambda i: (i, 0)),
            pl.BlockSpec((EOUT, H), lambda i: (0, 0)),
            pl.BlockSpec((1, H), lambda i: (0, 0)),
            pl.BlockSpec((H, VOUT), lambda i: (0, 0)),
            pl.BlockSpec((1, VOUT), lambda i: (0, 0)),
            pl.BlockSpec((1, VOUT), lambda i: (0, 0)),
            pl.BlockSpec((1, VOUT), lambda i: (0, 0)),
        ],
        out_specs=[
            pl.BlockSpec((bn, VOUT), lambda i: (i, 0)),
            pl.BlockSpec((1, VOUT), lambda i: (0, 0)),
        ],
        out_shape=[jax.ShapeDtypeStruct((N, VOUT), jnp.float32),
                   jax.ShapeDtypeStruct((1, VOUT), jnp.float32)],
    )(xn, a0, a1, wna, c2, nw2, nb2, ng, nbt)


def _global_body(u_ref, xsum_ref, esum_ref, gu_ref, gx_ref, ge_ref, gb1_ref,
                 gw2_ref, gb2_ref, gg_ref, gbt_ref, nu_ref):
    h = jnp.dot(u_ref[...], gu_ref[...], preferred_element_type=jnp.float32)
    h = h + jnp.dot(xsum_ref[...] * (1.0 / N), gx_ref[...],
                    preferred_element_type=jnp.float32)
    h = h + jnp.dot(esum_ref[...] * (1.0 / E), ge_ref[...],
                    preferred_element_type=jnp.float32)
    h = jnp.maximum(h + gb1_ref[...], 0.0)
    h = jnp.dot(h, gw2_ref[...], preferred_element_type=jnp.float32) + gb2_ref[...]
    h = jnp.maximum(h, 0.0)
    nu_ref[...] = _ln(h, gg_ref[...], gbt_ref[...])


def _global_mlp(u, xsum, esum, gu, gx, ge, gb1, gw2, gb2, gg, gbt):
    return pl.pallas_call(
        _global_body,
        out_shape=jax.ShapeDtypeStruct((1, UOUT), jnp.float32),
    )(u, xsum, esum, gu, gx, ge, gb1, gw2, gb2, gg, gbt)


def kernel(x, edge_index, edge_attr, u, v_indices, e_indices,
           ew1, eb1, ew2, eb2, eg, ebt,
           nw1, nb1, nw2, nb2, ng, nbt,
           gw1, gb1, gw2, gb2, gg, gbt):
    row = edge_index[0]
    col = edge_index[1]
    row2d = row.reshape(NCHUNKS, CHUNK)
    col2d = col.reshape(NCHUNKS, CHUNK)
    pad = jnp.full(((NCH_PAD - NCHUNKS) * CHUNK,), N, jnp.int32)
    rowpad2d = jnp.concatenate([row, pad]).reshape(NCH_PAD, CHUNK)

    bf = jnp.bfloat16
    wrcx = jnp.concatenate([ew1[:DV], ew1[DV:2 * DV], nw1[:DV]], axis=1).astype(bf)
    wu2 = jnp.concatenate([ew1[2 * DV + DE:], nw1[DV + EOUT:]], axis=1)
    b2 = jnp.concatenate([eb1, nb1]).reshape(1, -1)
    wna = nw1[DV:DV + EOUT].astype(bf)
    gu_w = gw1[:DU]
    gx_w = gw1[DU:DU + VOUT]
    ge_w = gw1[DU + VOUT:]
    wa = ew1[2 * DV:2 * DV + DE].astype(bf)

    r2 = lambda v: v.reshape(1, -1)

    xr, xc, xn, c1, c2 = _precompute(x, wrcx, u, wu2, b2)

    gr, gc = _gather(xr, xc, row2d, col2d)
    new_e, esum = _edge_mlp(gr, gc, edge_attr, wa, c1,
                            ew2.astype(bf), r2(eb2), r2(eg), r2(ebt))
    aggp = _scatter(new_e, rowpad2d)
    new_x, xsum = _node_mlp(xn, aggp[0], aggp[1], wna, c2,
                            nw2.astype(bf), r2(nb2), r2(ng), r2(nbt))
    new_u = _global_mlp(u, xsum, esum, gu_w, gx_w, ge_w, r2(gb1),
                        gw2, r2(gb2), r2(gg), r2(gbt))
    return (new_x, new_e, new_u)


# speedup vs baseline: 1.1214x; 1.0998x over previous
import functools

import jax
import jax.numpy as jnp
from jax import lax
from jax.experimental import pallas as pl
from jax.experimental.pallas import tpu as pltpu
from jax.experimental.pallas import tpu_sc as plsc

N = 10000
E = 160000
DV = 256
DE = 16
DU = 32
H = 256
VOUT = 256
EOUT = 128
UOUT = 32

CHUNK = 128
NW = 32
N_PAD = 10240
ROWS_PER_TILE = N_PAD // 16

NCHUNKS = E // CHUNK
GITER = (NCHUNKS + 15) // 16
SITER = (NCHUNKS + NW - 1) // NW
NCH_PAD = SITER * NW
TLOAD = N // 1000

_mesh = plsc.VectorSubcoreMesh(core_axis_name="c", subcore_axis_name="s")


def _ln(h, g, b):
    mu = jnp.mean(h, axis=-1, keepdims=True)
    d = h - mu
    var = jnp.mean(d * d, axis=-1, keepdims=True)
    return d * jax.lax.rsqrt(var + 1e-5) * g + b


def _pack_bf16(block):
    b = jax.lax.bitcast_convert_type(block, jnp.uint32)
    r = lambda v: (v + 0x7FFF + ((v >> 16) & 1)) >> 16
    return r(b[:, :H // 2]) | (r(b[:, H // 2:]) << 16)


def _unpack_bf16(packed):
    left = jax.lax.bitcast_convert_type(packed << 16, jnp.float32)
    right = jax.lax.bitcast_convert_type(packed & jnp.uint32(0xFFFF0000),
                                         jnp.float32)
    return left, right


def _pre_body(x_ref, w_ref, u_ref, wu2_ref, b2_ref, xr_ref, xc_ref, xn_ref,
              c1_ref, c2_ref):
    prod = jnp.dot(x_ref[...].astype(jnp.bfloat16), w_ref[...],
                   preferred_element_type=jnp.float32)
    xr_ref[...] = _pack_bf16(prod[:, :H])
    xc_ref[...] = _pack_bf16(prod[:, H:2 * H])
    xn_ref[...] = prod[:, 2 * H:]

    @pl.when(pl.program_id(0) == 0)
    def _():
        cu = jnp.dot(u_ref[...], wu2_ref[...],
                     preferred_element_type=jnp.float32) + b2_ref[...]
        c1_ref[...] = cu[:, :H]
        c2_ref[...] = cu[:, H:]


def _precompute(x, wrcx, u, wu2, b2):
    bn = 1000
    return pl.pallas_call(
        _pre_body,
        grid=(N // bn,),
        in_specs=[
            pl.BlockSpec((bn, DV), lambda i: (i, 0)),
            pl.BlockSpec((DV, 3 * H), lambda i: (0, 0)),
            pl.BlockSpec((1, DU), lambda i: (0, 0)),
            pl.BlockSpec((DU, 2 * H), lambda i: (0, 0)),
            pl.BlockSpec((1, 2 * H), lambda i: (0, 0)),
        ],
        out_specs=[
            pl.BlockSpec((bn, H // 2), lambda i: (i, 0)),
            pl.BlockSpec((bn, H // 2), lambda i: (i, 0)),
            pl.BlockSpec((bn, H), lambda i: (i, 0)),
            pl.BlockSpec((1, H), lambda i: (0, 0)),
            pl.BlockSpec((1, H), lambda i: (0, 0)),
        ],
        out_shape=[jax.ShapeDtypeStruct((N, H // 2), jnp.uint32),
                   jax.ShapeDtypeStruct((N, H // 2), jnp.uint32),
                   jax.ShapeDtypeStruct((N, H), jnp.float32),
                   jax.ShapeDtypeStruct((1, H), jnp.float32),
                   jax.ShapeDtypeStruct((1, H), jnp.float32)],
    )(x, wrcx, u, wu2, b2)


def _gather_body(xr_hbm, xc_hbm, row_hbm, col_hbm, gr_hbm, gc_hbm,
                 idxr_all, idxc_all, bufr_a, bufc_a, bufr_b, bufc_b,
                 sem_a, sem_b):
    wid = lax.axis_index("s") * 2 + lax.axis_index("c")
    base = wid * SITER

    pltpu.sync_copy(row_hbm.at[pl.ds(base, SITER)], idxr_all)
    pltpu.sync_copy(col_hbm.at[pl.ds(base, SITER)], idxc_all)

    def fire(j, br, bc, sem):
        pltpu.async_copy(xr_hbm.at[idxr_all.at[j]], br, sem)
        pltpu.async_copy(xc_hbm.at[idxc_all.at[j]], bc, sem)

    def drain(br, bc, sem):
        pltpu.make_async_copy(xr_hbm.at[idxr_all.at[0]], br, sem).wait()
        pltpu.make_async_copy(xc_hbm.at[idxc_all.at[0]], bc, sem).wait()

    def wb(j, br, bc):
        c = jnp.minimum(base + j, NCHUNKS - 1)
        pltpu.sync_copy(br, gr_hbm.at[pl.ds(c * CHUNK, CHUNK)])
        pltpu.sync_copy(bc, gc_hbm.at[pl.ds(c * CHUNK, CHUNK)])

    fire(0, bufr_a, bufc_a, sem_a)
    fire(1, bufr_b, bufc_b, sem_b)

    def step(p, _):
        ja = 2 * p
        drain(bufr_a, bufc_a, sem_a)
        wb(ja, bufr_a, bufc_a)

        @pl.when(ja + 2 < SITER)
        def _():
            fire(ja + 2, bufr_a, bufc_a, sem_a)
        drain(bufr_b, bufc_b, sem_b)
        wb(ja + 1, bufr_b, bufc_b)

        @pl.when(ja + 3 < SITER)
        def _():
            fire(ja + 3, bufr_b, bufc_b, sem_b)
        return 0

    lax.fori_loop(0, SITER // 2, step, 0)


@functools.partial(
    pl.kernel,
    out_type=[jax.ShapeDtypeStruct((E, H // 2), jnp.uint32),
              jax.ShapeDtypeStruct((E, H // 2), jnp.uint32)],
    mesh=_mesh,
    scratch_types=[
        pltpu.VMEM((SITER, CHUNK), jnp.int32),
        pltpu.VMEM((SITER, CHUNK), jnp.int32),
        pltpu.VMEM((CHUNK, H // 2), jnp.uint32),
        pltpu.VMEM((CHUNK, H // 2), jnp.uint32),
        pltpu.VMEM((CHUNK, H // 2), jnp.uint32),
        pltpu.VMEM((CHUNK, H // 2), jnp.uint32),
        pltpu.SemaphoreType.DMA,
        pltpu.SemaphoreType.DMA,
    ],
)
def _gather(*args):
    _gather_body(*args)


def _edge_body(gr_ref, gc_ref, ea_ref, wa_ref, c1_ref,
               ew2_ref, eb2_ref, eg_ref, ebt_ref, ne_ref, esum_ref):
    i = pl.program_id(0)
    rl, rr = _unpack_bf16(gr_ref[...])
    cl, cr = _unpack_bf16(gc_ref[...])
    h = jnp.concatenate([rl + cl, rr + cr], axis=1) + c1_ref[...]
    h = h + jnp.dot(ea_ref[...].astype(jnp.bfloat16), wa_ref[...],
                    preferred_element_type=jnp.float32)
    h = jnp.maximum(h, 0.0)
    h = jnp.dot(h.astype(jnp.bfloat16), ew2_ref[...],
                preferred_element_type=jnp.float32) + eb2_ref[...]
    h = jnp.maximum(h, 0.0)
    ne = _ln(h, eg_ref[...], ebt_ref[...])
    ne_ref[...] = ne

    @pl.when(i == 0)
    def _():
        esum_ref[...] = jnp.zeros_like(esum_ref)
    esum_ref[...] += jnp.sum(ne, axis=0, keepdims=True)


def _edge_mlp(gr, gc, ea, wa, c1, ew2, eb2, eg, ebt):
    be = 4000
    return pl.pallas_call(
        _edge_body,
        grid=(E // be,),
        in_specs=[
            pl.BlockSpec((be, H // 2), lambda i: (i, 0)),
            pl.BlockSpec((be, H // 2), lambda i: (i, 0)),
            pl.BlockSpec((be, DE), lambda i: (i, 0)),
            pl.BlockSpec((DE, H), lambda i: (0, 0)),
            pl.BlockSpec((1, H), lambda i: (0, 0)),
            pl.BlockSpec((H, EOUT), lambda i: (0, 0)),
            pl.BlockSpec((1, EOUT), lambda i: (0, 0)),
            pl.BlockSpec((1, EOUT), lambda i: (0, 0)),
            pl.BlockSpec((1, EOUT), lambda i: (0, 0)),
        ],
        out_specs=[
            pl.BlockSpec((be, EOUT), lambda i: (i, 0)),
            pl.BlockSpec((1, EOUT), lambda i: (0, 0)),
        ],
        out_shape=[jax.ShapeDtypeStruct((E, EOUT), jnp.float32),
                   jax.ShapeDtypeStruct((1, EOUT), jnp.float32)],
    )(gr, gc, ea, wa, c1, ew2, eb2, eg, ebt)


def _scatter_body(ne_hbm, row_hbm, agg_hbm, acc, idx_a, idx_b,
                  buf_a, buf_b, sem_a, sem_b):
    cid = lax.axis_index("c")
    sid = lax.axis_index("s")
    wid = sid * 2 + cid
    npass = ROWS_PER_TILE // CHUNK

    zeros16 = jnp.zeros((16,), jnp.float32)

    def zstep(r, _):
        for j in range(EOUT // 16):
            buf_a[r, pl.ds(j * 16, 16)] = zeros16
        return 0

    lax.fori_loop(0, CHUNK, zstep, 0)
    for p in range(npass):
        pltpu.sync_copy(
            buf_a, acc.at[pl.ds(sid * ROWS_PER_TILE + p * CHUNK, CHUNK)])
    plsc.subcore_barrier()

    def load(c, idx, buf, sem):
        pltpu.sync_copy(row_hbm.at[c], idx)
        c_data = jnp.minimum(c, NCHUNKS - 1)
        pltpu.async_copy(ne_hbm.at[pl.ds(c_data * CHUNK, CHUNK)], buf, sem)

    load(wid, idx_a, buf_a, sem_a)
    load(wid + NW, idx_b, buf_b, sem_b)

    def step(p, _):
        ca = wid + 2 * p * NW
        pltpu.make_async_copy(ne_hbm.at[pl.ds(0, CHUNK)], buf_a, sem_a).wait()
        pltpu.sync_copy(buf_a, acc.at[idx_a], add=True)

        @pl.when(2 * p + 2 < SITER)
        def _():
            load(ca + 2 * NW, idx_a, buf_a, sem_a)
        pltpu.make_async_copy(ne_hbm.at[pl.ds(0, CHUNK)], buf_b, sem_b).wait()
        pltpu.sync_copy(buf_b, acc.at[idx_b], add=True)

        @pl.when(2 * p + 3 < SITER)
        def _():
            load(ca + 3 * NW, idx_b, buf_b, sem_b)
        return 0

    lax.fori_loop(0, SITER // 2, step, 0)
    plsc.subcore_barrier()
    for p in range(npass):
        base = sid * ROWS_PER_TILE + p * CHUNK
        pltpu.sync_copy(acc.at[pl.ds(base, CHUNK)], buf_a)
        pltpu.sync_copy(buf_a, agg_hbm.at[cid, pl.ds(base, CHUNK)])


@functools.partial(
    pl.kernel,
    out_type=jax.ShapeDtypeStruct((2, N_PAD, EOUT), jnp.float32),
    mesh=_mesh,
    scratch_types=[
        pltpu.VMEM_SHARED((N_PAD, EOUT), jnp.float32),
        pltpu.VMEM((CHUNK,), jnp.int32),
        pltpu.VMEM((CHUNK,), jnp.int32),
        pltpu.VMEM((CHUNK, EOUT), jnp.float32),
        pltpu.VMEM((CHUNK, EOUT), jnp.float32),
        pltpu.SemaphoreType.DMA,
        pltpu.SemaphoreType.DMA,
    ],
)
def _scatter(*args):
    _scatter_body(*args)


def _node_body(xn_ref, a0_ref, a1_ref, wna_ref, c2_ref,
               nw2_ref, nb2_ref, ng_ref, nbt_ref, nx_ref, xsum_ref):
    i = pl.program_id(0)
    agg = (a0_ref[...] + a1_ref[...]).astype(jnp.bfloat16)
    h = xn_ref[...] + jnp.dot(agg, wna_ref[...],
                              preferred_element_type=jnp.float32) + c2_ref[...]
    h = jnp.maximum(h, 0.0)
    h = jnp.dot(h.astype(jnp.bfloat16), nw2_ref[...],
                preferred_element_type=jnp.float32) + nb2_ref[...]
    h = jnp.maximum(h, 0.0)
    nx = _ln(h, ng_ref[...], nbt_ref[...])
    nx_ref[...] = nx

    @pl.when(i == 0)
    def _():
        xsum_ref[...] = jnp.zeros_like(xsum_ref)
    xsum_ref[...] += jnp.sum(nx, axis=0, keepdims=True)


def _node_mlp(xn, a0, a1, wna, c2, nw2, nb2, ng, nbt):
    bn = 1000
    return pl.pallas_call(
        _node_body,
        grid=(N // bn,),
        in_specs=[
            pl.BlockSpec((bn, H), lambda i: (i, 0)),
            pl.BlockSpec((bn, EOUT), lambda i: (i, 0)),
            pl.BlockSpec((bn, EOUT), lambda i: (i, 0)),
            pl.BlockSpec((EOUT, H), lambda i: (0, 0)),
            pl.BlockSpec((1, H), lambda i: (0, 0)),
            pl.BlockSpec((H, VOUT), lambda i: (0, 0)),
            pl.BlockSpec((1, VOUT), lambda i: (0, 0)),
            pl.BlockSpec((1, VOUT), lambda i: (0, 0)),
            pl.BlockSpec((1, VOUT), lambda i: (0, 0)),
        ],
        out_specs=[
            pl.BlockSpec((bn, VOUT), lambda i: (i, 0)),
            pl.BlockSpec((1, VOUT), lambda i: (0, 0)),
        ],
        out_shape=[jax.ShapeDtypeStruct((N, VOUT), jnp.float32),
                   jax.ShapeDtypeStruct((1, VOUT), jnp.float32)],
    )(xn, a0, a1, wna, c2, nw2, nb2, ng, nbt)


def _global_body(u_ref, xsum_ref, esum_ref, gu_ref, gx_ref, ge_ref, gb1_ref,
                 gw2_ref, gb2_ref, gg_ref, gbt_ref, nu_ref):
    h = jnp.dot(u_ref[...], gu_ref[...], preferred_element_type=jnp.float32)
    h = h + jnp.dot(xsum_ref[...] * (1.0 / N), gx_ref[...],
                    preferred_element_type=jnp.float32)
    h = h + jnp.dot(esum_ref[...] * (1.0 / E), ge_ref[...],
                    preferred_element_type=jnp.float32)
    h = jnp.maximum(h + gb1_ref[...], 0.0)
    h = jnp.dot(h, gw2_ref[...], preferred_element_type=jnp.float32) + gb2_ref[...]
    h = jnp.maximum(h, 0.0)
    nu_ref[...] = _ln(h, gg_ref[...], gbt_ref[...])


def _global_mlp(u, xsum, esum, gu, gx, ge, gb1, gw2, gb2, gg, gbt):
    return pl.pallas_call(
        _global_body,
        out_shape=jax.ShapeDtypeStruct((1, UOUT), jnp.float32),
    )(u, xsum, esum, gu, gx, ge, gb1, gw2, gb2, gg, gbt)


def kernel(x, edge_index, edge_attr, u, v_indices, e_indices,
           ew1, eb1, ew2, eb2, eg, ebt,
           nw1, nb1, nw2, nb2, ng, nbt,
           gw1, gb1, gw2, gb2, gg, gbt):
    row = edge_index[0]
    col = edge_index[1]
    nrep = NCH_PAD - NCHUNKS
    gpad = lambda v: jnp.concatenate(
        [v, jnp.tile(v[-CHUNK:], nrep)]).reshape(NCH_PAD, CHUNK)
    row2d = gpad(row)
    col2d = gpad(col)
    pad = jnp.full((nrep * CHUNK,), N, jnp.int32)
    rowpad2d = jnp.concatenate([row, pad]).reshape(NCH_PAD, CHUNK)

    bf = jnp.bfloat16
    wrcx = jnp.concatenate([ew1[:DV], ew1[DV:2 * DV], nw1[:DV]], axis=1).astype(bf)
    wu2 = jnp.concatenate([ew1[2 * DV + DE:], nw1[DV + EOUT:]], axis=1)
    b2 = jnp.concatenate([eb1, nb1]).reshape(1, -1)
    wna = nw1[DV:DV + EOUT].astype(bf)
    gu_w = gw1[:DU]
    gx_w = gw1[DU:DU + VOUT]
    ge_w = gw1[DU + VOUT:]
    wa = ew1[2 * DV:2 * DV + DE].astype(bf)

    r2 = lambda v: v.reshape(1, -1)

    xr, xc, xn, c1, c2 = _precompute(x, wrcx, u, wu2, b2)

    gr, gc = _gather(xr, xc, row2d, col2d)
    new_e, esum = _edge_mlp(gr, gc, edge_attr, wa, c1,
                            ew2.astype(bf), r2(eb2), r2(eg), r2(ebt))
    aggp = _scatter(new_e, rowpad2d)
    new_x, xsum = _node_mlp(xn, aggp[0], aggp[1], wna, c2,
                            nw2.astype(bf), r2(nb2), r2(ng), r2(nbt))
    new_u = _global_mlp(u, xsum, esum, gu_w, gx_w, ge_w, r2(gb1),
                        gw2, r2(gb2), r2(gg), r2(gbt))
    return (new_x, new_e, new_u)


# speedup vs baseline: 1.1457x; 1.0217x over previous
import functools

import jax
import jax.numpy as jnp
from jax import lax
from jax.experimental import pallas as pl
from jax.experimental.pallas import tpu as pltpu
from jax.experimental.pallas import tpu_sc as plsc

N = 10000
E = 160000
DV = 256
DE = 16
DU = 32
H = 256
VOUT = 256
EOUT = 128
UOUT = 32

CHUNK = 128
NW = 32
N_PAD = 10240
ROWS_PER_TILE = N_PAD // 16

NCHUNKS = E // CHUNK
GITER = (NCHUNKS + 15) // 16
SITER = (NCHUNKS + NW - 1) // NW
NCH_PAD = SITER * NW
TLOAD = N // 1000

_mesh = plsc.VectorSubcoreMesh(core_axis_name="c", subcore_axis_name="s")


def _ln(h, g, b):
    mu = jnp.mean(h, axis=-1, keepdims=True)
    d = h - mu
    var = jnp.mean(d * d, axis=-1, keepdims=True)
    return d * jax.lax.rsqrt(var + 1e-5) * g + b


def _pack_bf16(block):
    b = jax.lax.bitcast_convert_type(block, jnp.uint32)
    r = lambda v: (v + 0x7FFF + ((v >> 16) & 1)) >> 16
    return r(b[:, :H // 2]) | (r(b[:, H // 2:]) << 16)


def _unpack_bf16(packed):
    left = jax.lax.bitcast_convert_type(packed << 16, jnp.float32)
    right = jax.lax.bitcast_convert_type(packed & jnp.uint32(0xFFFF0000),
                                         jnp.float32)
    return left, right


def _pre_body(x_ref, w_ref, u_ref, wu2_ref, b2_ref, xr_ref, xc_ref, xn_ref,
              c1_ref, c2_ref):
    prod = jnp.dot(x_ref[...].astype(jnp.bfloat16), w_ref[...],
                   preferred_element_type=jnp.float32)
    xr_ref[...] = _pack_bf16(prod[:, :H])
    xc_ref[...] = _pack_bf16(prod[:, H:2 * H])
    xn_ref[...] = prod[:, 2 * H:]

    @pl.when(pl.program_id(0) == 0)
    def _():
        cu = jnp.dot(u_ref[...], wu2_ref[...],
                     preferred_element_type=jnp.float32) + b2_ref[...]
        c1_ref[...] = cu[:, :H]
        c2_ref[...] = cu[:, H:]


def _precompute(x, wrcx, u, wu2, b2):
    bn = 1000
    return pl.pallas_call(
        _pre_body,
        grid=(N // bn,),
        in_specs=[
            pl.BlockSpec((bn, DV), lambda i: (i, 0)),
            pl.BlockSpec((DV, 3 * H), lambda i: (0, 0)),
            pl.BlockSpec((1, DU), lambda i: (0, 0)),
            pl.BlockSpec((DU, 2 * H), lambda i: (0, 0)),
            pl.BlockSpec((1, 2 * H), lambda i: (0, 0)),
        ],
        out_specs=[
            pl.BlockSpec((bn, H // 2), lambda i: (i, 0)),
            pl.BlockSpec((bn, H // 2), lambda i: (i, 0)),
            pl.BlockSpec((bn, H), lambda i: (i, 0)),
            pl.BlockSpec((1, H), lambda i: (0, 0)),
            pl.BlockSpec((1, H), lambda i: (0, 0)),
        ],
        out_shape=[jax.ShapeDtypeStruct((N, H // 2), jnp.uint32),
                   jax.ShapeDtypeStruct((N, H // 2), jnp.uint32),
                   jax.ShapeDtypeStruct((N, H), jnp.float32),
                   jax.ShapeDtypeStruct((1, H), jnp.float32),
                   jax.ShapeDtypeStruct((1, H), jnp.float32)],
    )(x, wrcx, u, wu2, b2)


def _gather_body(xr_hbm, xc_hbm, row_hbm, col_hbm, gr_hbm, gc_hbm,
                 idxr_all, idxc_all, bufr_a, bufc_a, bufr_b, bufc_b,
                 sem_a, sem_b):
    wid = lax.axis_index("s") * 2 + lax.axis_index("c")
    base = wid * SITER

    pltpu.sync_copy(row_hbm.at[pl.ds(base, SITER)], idxr_all)
    pltpu.sync_copy(col_hbm.at[pl.ds(base, SITER)], idxc_all)

    def fire(j, br, bc, sem):
        pltpu.async_copy(xr_hbm.at[idxr_all.at[j]], br, sem)
        pltpu.async_copy(xc_hbm.at[idxc_all.at[j]], bc, sem)

    def drain(br, bc, sem):
        pltpu.make_async_copy(xr_hbm.at[idxr_all.at[0]], br, sem).wait()
        pltpu.make_async_copy(xc_hbm.at[idxc_all.at[0]], bc, sem).wait()

    def wb(j, br, bc):
        c = jnp.minimum(base + j, NCHUNKS - 1)
        pltpu.sync_copy(br, gr_hbm.at[pl.ds(c * CHUNK, CHUNK)])
        pltpu.sync_copy(bc, gc_hbm.at[pl.ds(c * CHUNK, CHUNK)])

    fire(0, bufr_a, bufc_a, sem_a)
    fire(1, bufr_b, bufc_b, sem_b)

    def step(p, _):
        ja = 2 * p
        drain(bufr_a, bufc_a, sem_a)
        wb(ja, bufr_a, bufc_a)

        @pl.when(ja + 2 < SITER)
        def _():
            fire(ja + 2, bufr_a, bufc_a, sem_a)
        drain(bufr_b, bufc_b, sem_b)
        wb(ja + 1, bufr_b, bufc_b)

        @pl.when(ja + 3 < SITER)
        def _():
            fire(ja + 3, bufr_b, bufc_b, sem_b)
        return 0

    lax.fori_loop(0, SITER // 2, step, 0)


@functools.partial(
    pl.kernel,
    out_type=[jax.ShapeDtypeStruct((E, H // 2), jnp.uint32),
              jax.ShapeDtypeStruct((E, H // 2), jnp.uint32)],
    mesh=_mesh,
    scratch_types=[
        pltpu.VMEM((SITER, CHUNK), jnp.int32),
        pltpu.VMEM((SITER, CHUNK), jnp.int32),
        pltpu.VMEM((CHUNK, H // 2), jnp.uint32),
        pltpu.VMEM((CHUNK, H // 2), jnp.uint32),
        pltpu.VMEM((CHUNK, H // 2), jnp.uint32),
        pltpu.VMEM((CHUNK, H // 2), jnp.uint32),
        pltpu.SemaphoreType.DMA,
        pltpu.SemaphoreType.DMA,
    ],
)
def _gather(*args):
    _gather_body(*args)


def _edge_body(gr_ref, gc_ref, ea_ref, wa_ref, c1_ref,
               ew2_ref, eb2_ref, eg_ref, ebt_ref, ne_ref, esum_ref):
    i = pl.program_id(0)
    rl, rr = _unpack_bf16(gr_ref[...])
    cl, cr = _unpack_bf16(gc_ref[...])
    h = jnp.concatenate([rl + cl, rr + cr], axis=1) + c1_ref[...]
    h = h + jnp.dot(ea_ref[...].astype(jnp.bfloat16), wa_ref[...],
                    preferred_element_type=jnp.float32)
    h = jnp.maximum(h, 0.0)
    h = jnp.dot(h.astype(jnp.bfloat16), ew2_ref[...],
                preferred_element_type=jnp.float32) + eb2_ref[...]
    h = jnp.maximum(h, 0.0)
    ne = _ln(h, eg_ref[...], ebt_ref[...])
    ne_ref[...] = ne

    @pl.when(i == 0)
    def _():
        esum_ref[...] = jnp.zeros_like(esum_ref)
    esum_ref[...] += jnp.sum(ne, axis=0, keepdims=True)


def _edge_mlp(gr, gc, ea, wa, c1, ew2, eb2, eg, ebt):
    be = 4000
    return pl.pallas_call(
        _edge_body,
        grid=(E // be,),
        in_specs=[
            pl.BlockSpec((be, H // 2), lambda i: (i, 0)),
            pl.BlockSpec((be, H // 2), lambda i: (i, 0)),
            pl.BlockSpec((be, DE), lambda i: (i, 0)),
            pl.BlockSpec((DE, H), lambda i: (0, 0)),
            pl.BlockSpec((1, H), lambda i: (0, 0)),
            pl.BlockSpec((H, EOUT), lambda i: (0, 0)),
            pl.BlockSpec((1, EOUT), lambda i: (0, 0)),
            pl.BlockSpec((1, EOUT), lambda i: (0, 0)),
            pl.BlockSpec((1, EOUT), lambda i: (0, 0)),
        ],
        out_specs=[
            pl.BlockSpec((be, EOUT), lambda i: (i, 0)),
            pl.BlockSpec((1, EOUT), lambda i: (0, 0)),
        ],
        out_shape=[jax.ShapeDtypeStruct((E, EOUT), jnp.float32),
                   jax.ShapeDtypeStruct((1, EOUT), jnp.float32)],
    )(gr, gc, ea, wa, c1, ew2, eb2, eg, ebt)


def _scatter_body(ne_hbm, row_hbm, agg_hbm, acc, idx_all,
                  buf_a, buf_b, sem_a, sem_b):
    cid = lax.axis_index("c")
    sid = lax.axis_index("s")
    wid = sid * 2 + cid
    npass = ROWS_PER_TILE // CHUNK

    zeros16 = jnp.zeros((16,), jnp.float32)

    def zstep(r, _):
        for j in range(EOUT // 16):
            buf_a[r, pl.ds(j * 16, 16)] = zeros16
        return 0

    lax.fori_loop(0, CHUNK, zstep, 0)
    for p in range(npass):
        pltpu.sync_copy(
            buf_a, acc.at[pl.ds(sid * ROWS_PER_TILE + p * CHUNK, CHUNK)])
    plsc.subcore_barrier()

    base = wid * SITER
    pltpu.sync_copy(row_hbm.at[pl.ds(base, SITER)], idx_all)

    def load(j, buf, sem):
        c_data = jnp.minimum(base + j, NCHUNKS - 1)
        pltpu.async_copy(ne_hbm.at[pl.ds(c_data * CHUNK, CHUNK)], buf, sem)

    load(0, buf_a, sem_a)
    load(1, buf_b, sem_b)

    def step(p, _):
        ja = 2 * p
        pltpu.make_async_copy(ne_hbm.at[pl.ds(0, CHUNK)], buf_a, sem_a).wait()
        pltpu.sync_copy(buf_a, acc.at[idx_all.at[ja]], add=True)

        @pl.when(ja + 2 < SITER)
        def _():
            load(ja + 2, buf_a, sem_a)
        pltpu.make_async_copy(ne_hbm.at[pl.ds(0, CHUNK)], buf_b, sem_b).wait()
        pltpu.sync_copy(buf_b, acc.at[idx_all.at[ja + 1]], add=True)

        @pl.when(ja + 3 < SITER)
        def _():
            load(ja + 3, buf_b, sem_b)
        return 0

    lax.fori_loop(0, SITER // 2, step, 0)
    plsc.subcore_barrier()
    for p in range(npass):
        base = sid * ROWS_PER_TILE + p * CHUNK
        pltpu.sync_copy(acc.at[pl.ds(base, CHUNK)], buf_a)
        pltpu.sync_copy(buf_a, agg_hbm.at[cid, pl.ds(base, CHUNK)])


@functools.partial(
    pl.kernel,
    out_type=jax.ShapeDtypeStruct((2, N_PAD, EOUT), jnp.float32),
    mesh=_mesh,
    scratch_types=[
        pltpu.VMEM_SHARED((N_PAD, EOUT), jnp.float32),
        pltpu.VMEM((SITER, CHUNK), jnp.int32),
        pltpu.VMEM((CHUNK, EOUT), jnp.float32),
        pltpu.VMEM((CHUNK, EOUT), jnp.float32),
        pltpu.SemaphoreType.DMA,
        pltpu.SemaphoreType.DMA,
    ],
)
def _scatter(*args):
    _scatter_body(*args)


def _node_body(xn_ref, a0_ref, a1_ref, wna_ref, c2_ref,
               nw2_ref, nb2_ref, ng_ref, nbt_ref, nx_ref, xsum_ref):
    i = pl.program_id(0)
    agg = (a0_ref[...] + a1_ref[...]).astype(jnp.bfloat16)
    h = xn_ref[...] + jnp.dot(agg, wna_ref[...],
                              preferred_element_type=jnp.float32) + c2_ref[...]
    h = jnp.maximum(h, 0.0)
    h = jnp.dot(h.astype(jnp.bfloat16), nw2_ref[...],
                preferred_element_type=jnp.float32) + nb2_ref[...]
    h = jnp.maximum(h, 0.0)
    nx = _ln(h, ng_ref[...], nbt_ref[...])
    nx_ref[...] = nx

    @pl.when(i == 0)
    def _():
        xsum_ref[...] = jnp.zeros_like(xsum_ref)
    xsum_ref[...] += jnp.sum(nx, axis=0, keepdims=True)


def _node_mlp(xn, a0, a1, wna, c2, nw2, nb2, ng, nbt):
    bn = 1000
    return pl.pallas_call(
        _node_body,
        grid=(N // bn,),
        in_specs=[
            pl.BlockSpec((bn, H), lambda i: (i, 0)),
            pl.BlockSpec((bn, EOUT), lambda i: (i, 0)),
            pl.BlockSpec((bn, EOUT), lambda i: (i, 0)),
            pl.BlockSpec((EOUT, H), lambda i: (0, 0)),
            pl.BlockSpec((1, H), lambda i: (0, 0)),
            pl.BlockSpec((H, VOUT), lambda i: (0, 0)),
            pl.BlockSpec((1, VOUT), lambda i: (0, 0)),
            pl.BlockSpec((1, VOUT), lambda i: (0, 0)),
            pl.BlockSpec((1, VOUT), lambda i: (0, 0)),
        ],
        out_specs=[
            pl.BlockSpec((bn, VOUT), lambda i: (i, 0)),
            pl.BlockSpec((1, VOUT), lambda i: (0, 0)),
        ],
        out_shape=[jax.ShapeDtypeStruct((N, VOUT), jnp.float32),
                   jax.ShapeDtypeStruct((1, VOUT), jnp.float32)],
    )(xn, a0, a1, wna, c2, nw2, nb2, ng, nbt)


def _global_body(u_ref, xsum_ref, esum_ref, gu_ref, gx_ref, ge_ref, gb1_ref,
                 gw2_ref, gb2_ref, gg_ref, gbt_ref, nu_ref):
    h = jnp.dot(u_ref[...], gu_ref[...], preferred_element_type=jnp.float32)
    h = h + jnp.dot(xsum_ref[...] * (1.0 / N), gx_ref[...],
                    preferred_element_type=jnp.float32)
    h = h + jnp.dot(esum_ref[...] * (1.0 / E), ge_ref[...],
                    preferred_element_type=jnp.float32)
    h = jnp.maximum(h + gb1_ref[...], 0.0)
    h = jnp.dot(h, gw2_ref[...], preferred_element_type=jnp.float32) + gb2_ref[...]
    h = jnp.maximum(h, 0.0)
    nu_ref[...] = _ln(h, gg_ref[...], gbt_ref[...])


def _global_mlp(u, xsum, esum, gu, gx, ge, gb1, gw2, gb2, gg, gbt):
    return pl.pallas_call(
        _global_body,
        out_shape=jax.ShapeDtypeStruct((1, UOUT), jnp.float32),
    )(u, xsum, esum, gu, gx, ge, gb1, gw2, gb2, gg, gbt)


def kernel(x, edge_index, edge_attr, u, v_indices, e_indices,
           ew1, eb1, ew2, eb2, eg, ebt,
           nw1, nb1, nw2, nb2, ng, nbt,
           gw1, gb1, gw2, gb2, gg, gbt):
    row = edge_index[0]
    col = edge_index[1]
    nrep = NCH_PAD - NCHUNKS
    gpad = lambda v: jnp.concatenate(
        [v, jnp.tile(v[-CHUNK:], nrep)]).reshape(NCH_PAD, CHUNK)
    row2d = gpad(row)
    col2d = gpad(col)
    pad = jnp.full((nrep * CHUNK,), N, jnp.int32)
    rowpad2d = jnp.concatenate([row, pad]).reshape(NCH_PAD, CHUNK)

    bf = jnp.bfloat16
    wrcx = jnp.concatenate([ew1[:DV], ew1[DV:2 * DV], nw1[:DV]], axis=1).astype(bf)
    wu2 = jnp.concatenate([ew1[2 * DV + DE:], nw1[DV + EOUT:]], axis=1)
    b2 = jnp.concatenate([eb1, nb1]).reshape(1, -1)
    wna = nw1[DV:DV + EOUT].astype(bf)
    gu_w = gw1[:DU]
    gx_w = gw1[DU:DU + VOUT]
    ge_w = gw1[DU + VOUT:]
    wa = ew1[2 * DV:2 * DV + DE].astype(bf)

    r2 = lambda v: v.reshape(1, -1)

    xr, xc, xn, c1, c2 = _precompute(x, wrcx, u, wu2, b2)

    gr, gc = _gather(xr, xc, row2d, col2d)
    new_e, esum = _edge_mlp(gr, gc, edge_attr, wa, c1,
                            ew2.astype(bf), r2(eb2), r2(eg), r2(ebt))
    aggp = _scatter(new_e, rowpad2d)
    new_x, xsum = _node_mlp(xn, aggp[0], aggp[1], wna, c2,
                            nw2.astype(bf), r2(nb2), r2(ng), r2(nbt))
    new_u = _global_mlp(u, xsum, esum, gu_w, gx_w, ge_w, r2(gb1),
                        gw2, r2(gb2), r2(gg), r2(gbt))
    return (new_x, new_e, new_u)
